# Initial kernel scaffold; baseline (speedup 1.0000x reference)
import jax
import jax.numpy as jnp
from jax.experimental import pallas as pl


def kernel(x, om, table0, table1):
    raise NotImplementedError("write your pallas kernel here")


# speedup vs baseline: 45.8176x; 45.8176x over previous
import functools

import jax
import jax.numpy as jnp
import numpy as np
from jax import lax
from jax.experimental import pallas as pl
from jax.experimental.pallas import tpu as pltpu
from jax.experimental.pallas import tpu_sc as plsc

_K0, _K1 = 512, 512
_J = 6
_L = 1024
_TBL = _J * _L + 1
_TBLP = 6148
_B, _C, _KLEN = 2, 8, 131072
_ROWS = _K0 * _K1
_NTAP = _J * _J
_NW = 32
_WPB = _NW // _B
_SPW = _KLEN // _WPB
_S = 128
_NCHUNK = _SPW // _S
_NG = _S // 16

_MAGIC = np.float32(1.5 * 2 ** 23)
_SC0 = np.float32(_K0 / (2.0 * np.pi))
_SC1 = np.float32(_K1 / (2.0 * np.pi))


def _rne(x):
  return (x + _MAGIC) - _MAGIC


def _floor(x):
  r = _rne(x)
  return r - jnp.where(r > x, jnp.float32(1.0), jnp.float32(0.0))


def _tap_setup(omv, scale, tbl_ref):
  tm = omv * scale
  fl = _floor(tm - jnp.float32(_J / 2.0))
  koff = jnp.float32(1.0) + fl
  r = _rne((tm - koff) * jnp.float32(_L))
  dbase = r.astype(jnp.int32) + jnp.int32((_J * _L) // 2)
  gbase = koff.astype(jnp.int32)
  ar, ai, gi = [], [], []
  for j in range(_J):
    d = dbase - jnp.int32(_L * j)
    ar.append(plsc.load_gather(tbl_ref, [d]))
    ai.append(plsc.load_gather(tbl_ref, [d + jnp.int32(_TBLP)]))
    gi.append((gbase + jnp.int32(j)) & jnp.int32(_K0 - 1))
  return ar, ai, gi


def _body(xt, om2, prpi, tbl0, tbl1, out,
          tbl0_v, tbl1_v, idx_v, cr_v, ci_v, rows_v, out_v,
          om0_v, om1_v, pr_v, pi_v, gsem):
  wid = lax.axis_index("s") * 2 + lax.axis_index("c")
  b = wid // _WPB
  part = wid % _WPB
  wbase = part * _SPW
  brow = b * _ROWS

  pltpu.sync_copy(tbl0, tbl0_v)
  pltpu.sync_copy(tbl1, tbl1_v)

  iotai = lax.iota(jnp.int32, 16)

  def chunk_body(ch, carry):
    base = wbase + ch * _S
    pltpu.sync_copy(om2.at[2 * b, pl.ds(base, _S)], om0_v)
    pltpu.sync_copy(om2.at[2 * b + 1, pl.ds(base, _S)], om1_v)
    pltpu.sync_copy(prpi.at[2 * b, pl.ds(base, _S)], pr_v)
    pltpu.sync_copy(prpi.at[2 * b + 1, pl.ds(base, _S)], pi_v)

    def coef_body(g, c2):
      sl = pl.ds(g * 16, 16)
      om0v = om0_v[sl]
      om1v = om1_v[sl]
      a0r, a0i, i0 = _tap_setup(om0v, _SC0, tbl0_v)
      a1r, a1i, i1 = _tap_setup(om1v, _SC1, tbl1_v)
      for j0 in range(_J):
        row0 = (i0[j0] << jnp.int32(9)) + brow
        for j1 in range(_J):
          t = j0 * _J + j1
          cr = a0r[j0] * a1r[j1] - a0i[j0] * a1i[j1]
          ci = a0r[j0] * a1i[j1] + a0i[j0] * a1r[j1]
          idx_v[t, sl] = row0 + i1[j1]
          cr_v[t, sl] = cr
          ci_v[t, sl] = ci
      return c2

    lax.fori_loop(0, _NG, coef_body, 0)

    copies = [
        pltpu.async_copy(xt.at[idx_v.at[t]], rows_v.at[t], gsem)
        for t in range(_NTAP)
    ]
    for cp in copies:
      cp.wait()

    def acc_body(g, c3):
      sl = pl.ds(g * 16, 16)
      svec = g * 16 + iotai
      kr = [jnp.zeros((16,), jnp.float32) for _ in range(_C)]
      ki = [jnp.zeros((16,), jnp.float32) for _ in range(_C)]
      for t in range(_NTAP):
        tvec = jnp.full((16,), t, jnp.int32)
        crv = cr_v[t, sl]
        civ = ci_v[t, sl]
        for c in range(_C):
          gr = plsc.load_gather(
              rows_v, [tvec, svec, jnp.full((16,), c, jnp.int32)])
          gi = plsc.load_gather(
              rows_v, [tvec, svec, jnp.full((16,), c + _C, jnp.int32)])
          kr[c] = kr[c] + crv * gr - civ * gi
          ki[c] = ki[c] + crv * gi + civ * gr
      prv = pr_v[sl]
      piv = pi_v[sl]
      for c in range(_C):
        out_v[c, sl] = kr[c] * prv - ki[c] * piv
        out_v[c + _C, sl] = kr[c] * piv + ki[c] * prv
      return c3

    lax.fori_loop(0, _NG, acc_body, 0)

    pltpu.sync_copy(out_v, out.at[pl.ds(b * 16, 16), pl.ds(base, _S)])
    return carry

  lax.fori_loop(0, _NCHUNK, chunk_body, 0)


@jax.jit
def kernel(x, om, table0, table1):
  nb, nc = x.shape[0], x.shape[1]
  klen = om.shape[2]
  xt = jnp.transpose(x.reshape(nb, nc, 2, _ROWS), (0, 3, 2, 1))
  xt = xt.reshape(nb * _ROWS, 2 * nc)
  om2 = om.reshape(nb * 2, klen)
  ph = (om[:, 0, :] + om[:, 1, :]) * jnp.float32(128.0)
  prpi = jnp.stack([jnp.cos(ph), jnp.sin(ph)], axis=1).reshape(nb * 2, klen)
  t0 = jnp.pad(table0, ((0, 0), (0, _TBLP - _TBL))).reshape(-1)
  t1 = jnp.pad(table1, ((0, 0), (0, _TBLP - _TBL))).reshape(-1)

  mesh = plsc.VectorSubcoreMesh(core_axis_name="c", subcore_axis_name="s")
  out2 = pl.kernel(
      _body,
      out_type=jax.ShapeDtypeStruct((nb * 16, klen), jnp.float32),
      mesh=mesh,
      compiler_params=pltpu.CompilerParams(
          needs_layout_passes=False, use_tc_tiling_on_sc=False),
      scratch_types=[
          pltpu.VMEM((2 * _TBLP,), jnp.float32),
          pltpu.VMEM((2 * _TBLP,), jnp.float32),
          pltpu.VMEM((_NTAP, _S), jnp.int32),
          pltpu.VMEM((_NTAP, _S), jnp.float32),
          pltpu.VMEM((_NTAP, _S), jnp.float32),
          pltpu.VMEM((_NTAP, _S, 2 * _C), jnp.float32),
          pltpu.VMEM((2 * _C, _S), jnp.float32),
          pltpu.VMEM((_S,), jnp.float32),
          pltpu.VMEM((_S,), jnp.float32),
          pltpu.VMEM((_S,), jnp.float32),
          pltpu.VMEM((_S,), jnp.float32),
          pltpu.SemaphoreType.DMA,
      ],
  )(xt, om2, prpi, t0, t1)

  y = out2.reshape(nb, 2, nc, klen).transpose(0, 2, 1, 3)
  return y
